# Initial kernel scaffold; baseline (speedup 1.0000x reference)
import jax
import jax.numpy as jnp
from jax.experimental import pallas as pl


def kernel(inputs, input_logits, root_w, sum_w_0, sum_w_1, sum_w_2, sum_w_3, sum_w_4, sum_w_5):
    raise NotImplementedError("write your pallas kernel here")


# speedup vs baseline: 33.1526x; 33.1526x over previous
import functools

import jax
import jax.numpy as jnp
from jax import lax
from jax.experimental import pallas as pl
from jax.experimental.pallas import tpu as pltpu
from jax.experimental.pallas import tpu_sc as plsc

NUM_VARS = 64
K = 32
KK = K * K
NUM_CATS = 128
B = 256

_SC_NC = 2
_SC_NS = 16
_NW = _SC_NC * _SC_NS
_ROWS = B * NUM_VARS
_RPW = _ROWS // _NW


@functools.partial(
    pl.kernel,
    out_type=jax.ShapeDtypeStruct((_ROWS, K), jnp.float32),
    mesh=plsc.VectorSubcoreMesh(core_axis_name="c", subcore_axis_name="s"),
    scratch_types=[
        pltpu.VMEM((_RPW,), jnp.int32),
        pltpu.VMEM((_RPW, K), jnp.float32),
        pltpu.SemaphoreType.DMA,
    ],
    compiler_params=pltpu.CompilerParams(use_tc_tiling_on_sc=False),
)
def _sc_gather(table_hbm, idx_hbm, out_hbm, idx_v, rows_v, sem):
    wid = lax.axis_index("s") * _SC_NC + lax.axis_index("c")
    base = wid * _RPW
    pltpu.sync_copy(idx_hbm.at[pl.ds(base, _RPW)], idx_v)
    pltpu.async_copy(table_hbm.at[idx_v], rows_v, sem).wait()
    pltpu.sync_copy(rows_v, out_hbm.at[pl.ds(base, _RPW)])


def _tc_body(mars_ref, w0, w1, w2, w3, w4, w5, root_ref, out_ref,
             s0, s1, s2, s3, s4):
    w_refs = [w0, w1, w2, w3, w4, w5]
    out_bufs = [s0, s1, s2, s3, s4, None]
    in_buf = mars_ref
    final = None
    for lvl in range(6):
        rh = NUM_VARS >> (lvl + 1)
        w_ref = w_refs[lvl]
        for r in range(rh):
            left = in_buf[2 * r]
            right = in_buf[2 * r + 1]
            ml = jnp.max(left, axis=1, keepdims=True)
            mr = jnp.max(right, axis=1, keepdims=True)
            a = jnp.exp(left - ml)
            c = jnp.exp(right - mr)
            p = (a[:, :, None] * c[:, None, :]).reshape(B, KK)
            ew = jnp.exp(w_ref[r])
            o = lax.dot_general(p, ew, (((1,), (1,)), ((), ())),
                                preferred_element_type=jnp.float32)
            res = jnp.log(o + 1e-30) + ml + mr
            if lvl < 5:
                out_bufs[lvl][r] = res
            else:
                final = res
        in_buf = out_bufs[lvl]
    t = final + root_ref[0]
    m = jnp.max(t, axis=1, keepdims=True)
    out_ref[:] = jnp.log(jnp.sum(jnp.exp(t - m), axis=1, keepdims=True)) + m


def _tc_levels(mars0, ws, root_w):
    return pl.pallas_call(
        _tc_body,
        out_shape=jax.ShapeDtypeStruct((B, 1), jnp.float32),
        scratch_shapes=[
            pltpu.VMEM((NUM_VARS >> (l + 1), B, K), jnp.float32)
            for l in range(5)
        ],
    )(mars0, *ws, root_w)


def kernel(inputs, input_logits, root_w, sum_w_0, sum_w_1, sum_w_2,
           sum_w_3, sum_w_4, sum_w_5):
    table = jnp.transpose(input_logits, (0, 2, 1)).reshape(
        NUM_VARS * NUM_CATS, K)
    idx = (inputs.T.astype(jnp.int32)
           + jnp.arange(NUM_VARS, dtype=jnp.int32)[:, None] * NUM_CATS
           ).reshape(_ROWS)
    mars0 = _sc_gather(table, idx).reshape(NUM_VARS, B, K)
    ws = [sum_w_0, sum_w_1, sum_w_2, sum_w_3, sum_w_4, sum_w_5]
    return _tc_levels(mars0, ws, root_w)


# speedup vs baseline: 110.8620x; 3.3440x over previous
import functools

import jax
import jax.numpy as jnp
from jax import lax
from jax.experimental import pallas as pl
from jax.experimental.pallas import tpu as pltpu
from jax.experimental.pallas import tpu_sc as plsc

NUM_VARS = 64
K = 32
KK = K * K
NUM_CATS = 128
B = 256

_SC_NC = 2
_SC_NS = 16
_NW = _SC_NC * _SC_NS
_ROWS = B * NUM_VARS
_RPW = _ROWS // _NW


@functools.partial(
    pl.kernel,
    out_type=jax.ShapeDtypeStruct((_ROWS, K), jnp.float32),
    mesh=plsc.VectorSubcoreMesh(core_axis_name="c", subcore_axis_name="s"),
    scratch_types=[
        pltpu.VMEM((_RPW,), jnp.int32),
        pltpu.VMEM((_RPW, K), jnp.float32),
        pltpu.SemaphoreType.DMA,
    ],
    compiler_params=pltpu.CompilerParams(use_tc_tiling_on_sc=False),
)
def _sc_gather(table_hbm, idx_hbm, out_hbm, idx_v, rows_v, sem):
    wid = lax.axis_index("s") * _SC_NC + lax.axis_index("c")
    base = wid * _RPW
    pltpu.sync_copy(idx_hbm.at[pl.ds(base, _RPW)], idx_v)
    pltpu.async_copy(table_hbm.at[idx_v], rows_v, sem).wait()
    pltpu.sync_copy(rows_v, out_hbm.at[pl.ds(base, _RPW)])


def _tc_body(mars_ref, w0, w1, w2, w3, w4, w5, root_ref, out_ref,
             s0, s1, s2, s3, s4):
    w_refs = [w0, w1, w2, w3, w4, w5]
    out_bufs = [s0, s1, s2, s3, s4, None]
    in_buf = mars_ref
    final = None
    for lvl in range(6):
        rh = NUM_VARS >> (lvl + 1)
        w_ref = w_refs[lvl]
        for r in range(rh):
            left = in_buf[2 * r]
            right = in_buf[2 * r + 1]
            ml = jnp.max(left, axis=0, keepdims=True)
            mr = jnp.max(right, axis=0, keepdims=True)
            a = jnp.exp(left - ml)
            c = jnp.exp(right - mr)
            p = (a[:, None, :] * c[None, :, :]).reshape(KK, B)
            ew = jnp.exp(w_ref[r])
            o = lax.dot_general(ew, p, (((1,), (0,)), ((), ())),
                                preferred_element_type=jnp.float32)
            res = jnp.log(o + 1e-30) + ml + mr
            if lvl < 5:
                out_bufs[lvl][r] = res
            else:
                final = res
        in_buf = out_bufs[lvl]
    t = final + root_ref[:]
    m = jnp.max(t, axis=0, keepdims=True)
    out_ref[:] = jnp.log(jnp.sum(jnp.exp(t - m), axis=0, keepdims=True)) + m


def _tc_levels(mars0, ws, root_w):
    out = pl.pallas_call(
        _tc_body,
        out_shape=jax.ShapeDtypeStruct((1, B), jnp.float32),
        scratch_shapes=[
            pltpu.VMEM((NUM_VARS >> (l + 1), K, B), jnp.float32)
            for l in range(5)
        ],
    )(mars0, *ws, root_w)
    return out.reshape(B, 1)


def kernel(inputs, input_logits, root_w, sum_w_0, sum_w_1, sum_w_2,
           sum_w_3, sum_w_4, sum_w_5):
    table = jnp.transpose(input_logits, (0, 2, 1)).reshape(
        NUM_VARS * NUM_CATS, K)
    idx = (inputs.T.astype(jnp.int32)
           + jnp.arange(NUM_VARS, dtype=jnp.int32)[:, None] * NUM_CATS
           ).reshape(_ROWS)
    mars0 = jnp.transpose(
        _sc_gather(table, idx).reshape(NUM_VARS, B, K), (0, 2, 1))
    ws = [sum_w_0, sum_w_1, sum_w_2, sum_w_3, sum_w_4, sum_w_5]
    return _tc_levels(mars0, ws, root_w.reshape(K, 1))
